# Initial kernel scaffold; baseline (speedup 1.0000x reference)
import jax
import jax.numpy as jnp
from jax.experimental import pallas as pl


def kernel(x):
    raise NotImplementedError("write your pallas kernel here")


# speedup vs baseline: 7.1259x; 7.1259x over previous
import functools

import jax
import jax.numpy as jnp
from jax import lax
from jax.experimental import pallas as pl
from jax.experimental.pallas import tpu as pltpu
from jax.experimental.pallas import tpu_sc as plsc

N_ROWS = 2097152
F = N_ROWS * 8
NC = 2
NS = 16
NW = NC * NS
PER_W = F // NW
CHUNK = 16384
N_CHUNKS = PER_W // CHUNK
GROUPS = CHUNK // 128

MAGIC = 12582912.0


def _quantize_group(xs):
    f_ = [(xs[c] + MAGIC) - MAGIC for c in range(8)]
    d_ = [xs[c] - f_[c] for c in range(8)]
    a_ = [jnp.abs(d_[c]) for c in range(8)]
    neg = [d_[c] < 0.0 for c in range(8)]
    sgn = [jnp.where(neg[c], -1.0, 1.0) for c in range(8)]
    ind = [jnp.where(neg[c], 1.0, 0.0) for c in range(8)]

    def tree(op, vs):
        t0 = op(vs[0], vs[1]); t1 = op(vs[2], vs[3])
        t2 = op(vs[4], vs[5]); t3 = op(vs[6], vs[7])
        return op(op(t0, t1), op(t2, t3))

    m_a = tree(jnp.maximum, a_)
    m_n = tree(jnp.minimum, a_)
    sum_f = tree(jnp.add, f_)
    sum_i = tree(jnp.add, ind)
    sum_a = tree(jnp.add, a_)
    sq_a = tree(jnp.add, [d_[c] * d_[c] for c in range(8)])
    sq_b = (sq_a + 2.0) - sum_a

    k_a = jnp.zeros((16,), jnp.int32)
    k_b = jnp.zeros((16,), jnp.int32)
    fix_a = jnp.zeros((16,), jnp.float32)
    s_b = jnp.zeros((16,), jnp.float32)
    for c in range(7, -1, -1):
        ck_a = a_[c] == m_a
        ck_b = a_[c] == m_n
        k_a = jnp.where(ck_a, c, k_a)
        k_b = jnp.where(ck_b, c, k_b)
        fix_a = jnp.where(ck_a, sgn[c], fix_a)
        s_b = jnp.where(ck_b, sgn[c], s_b)
    fix_b = -s_b

    odd_a = lax.rem(sum_f, 2.0) != 0.0
    sum_fb = sum_f - sum_i
    odd_b = lax.rem(sum_fb, 2.0) != 0.0

    sq_ap = sq_a + jnp.where(odd_a, 1.0 - 2.0 * m_a, 0.0)
    sq_bp = sq_b + jnp.where(odd_b, 2.0 * m_n, 0.0)
    win_b = sq_bp < sq_ap

    k_w = jnp.where(win_b, k_b, k_a)
    val_a = jnp.where(odd_a, fix_a, 0.0)
    val_b = jnp.where(odd_b, fix_b, 0.0)
    val_w = jnp.where(win_b, val_b, val_a)

    ys = []
    for c in range(8):
        yc = f_[c] + jnp.where(win_b, 0.5 - ind[c], 0.0)
        yc = yc + jnp.where(k_w == c, val_w, 0.0)
        ys.append(yc)
    return ys


def _sc_body(x_hbm, out_hbm, buf_in, buf_out):
    cid = lax.axis_index("c")
    sid = lax.axis_index("s")
    wid = sid * NC + cid
    base_w = wid * PER_W
    st8 = lax.iota(jnp.int32, 16) * 8

    def chunk_body(i, carry):
        off = base_w + i * CHUNK
        pltpu.sync_copy(x_hbm.at[pl.ds(off, CHUNK)], buf_in)

        def group_body(g, c2):
            gb = st8 + g * 128
            xs = [plsc.load_gather(buf_in, [gb + c]) for c in range(8)]
            ys = _quantize_group(xs)
            for c in range(8):
                plsc.store_scatter(buf_out, [gb + c], ys[c])
            return c2

        lax.fori_loop(0, GROUPS, group_body, 0)
        pltpu.sync_copy(buf_out, out_hbm.at[pl.ds(off, CHUNK)])
        return carry

    lax.fori_loop(0, N_CHUNKS, chunk_body, 0)


@jax.jit
def _e8_quantize_flat(xf):
    run = functools.partial(
        pl.kernel,
        out_type=jax.ShapeDtypeStruct((F,), jnp.float32),
        mesh=plsc.VectorSubcoreMesh(core_axis_name="c", subcore_axis_name="s"),
        scratch_types=[
            pltpu.VMEM((CHUNK,), jnp.float32),
            pltpu.VMEM((CHUNK,), jnp.float32),
        ],
        compiler_params=pltpu.CompilerParams(needs_layout_passes=False),
    )
    return run(_sc_body)(xf)


def kernel(x):
    yf = _e8_quantize_flat(x.reshape(F))
    return yf.reshape(N_ROWS, 8)
